# baseline (speedup 1.0000x reference)
import jax
import jax.numpy as jnp
from jax import lax
from jax.experimental import pallas as pl
from jax.experimental.pallas import tpu as pltpu
from jax.experimental.pallas import tpu_sc as plsc

B = 64
E = 100000
R = 32
H = 16
C = 500000
EPS = 1e-6

CB = 16384
NMAIN = C // CB
CTAIL = C - NMAIN * CB
DEPTH = 4
NSPLIT = 4
RS = R // NSPLIT

SENT = B * E
CHUNK = 2048
LPAD = C + CHUNK + 16
NC = 2
NS = 16
LANES = 16
NSTARTS = 80


def _score(rc, ruleta_ref, sqrow_ref, w1g_ref, bb2_ref, w2t_ref, b2_ref):
    msgx = jnp.dot(ruleta_ref[...], rc,
                   preferred_element_type=jnp.float32)
    msg = msgx[0:H]
    d = msgx[H:H + 1] + EPS
    mu = msgx[H + 1:H + 2]
    xc = msg - mu
    var = jnp.dot(sqrow_ref[...], xc * xc,
                  preferred_element_type=jnp.float32)[0:1]
    inv = lax.rsqrt(var + 1e-5 * d * d)
    xhat = xc * inv
    z = jnp.dot(w1g_ref[...], xhat,
                preferred_element_type=jnp.float32)
    hdn = jnp.maximum(z + bb2_ref[...], 0.0)
    s = jnp.dot(w2t_ref[...], hdn,
                preferred_element_type=jnp.float32)[0:1]
    return s + b2_ref[...]


def _dense_body(rc_hbm, rc_tail_ref, csa_ref, csb_ref, ruleta_ref, sqrow_ref,
                w1g_ref, bb2_ref, w2t_ref, b2_ref, s_ref, sidx_ref,
                rc_buf, sems):
    i = pl.program_id(0)

    def issue(slot, blk):
        for sp in range(NSPLIT):
            pltpu.make_async_copy(
                rc_hbm.at[pl.ds(sp * RS, RS), pl.ds(blk * CB, CB)],
                rc_buf.at[slot, pl.ds(sp * RS, RS), :],
                sems.at[slot, sp]).start()

    @pl.when(i == 0)
    def _():
        for b in range(DEPTH - 1):
            issue(b, b)

    nxt = i + DEPTH - 1

    @pl.when(nxt < NMAIN)
    def _():
        issue(lax.rem(nxt, DEPTH), nxt)

    @pl.when(i < NMAIN)
    def _():
        slot = lax.rem(i, DEPTH)
        for sp in range(NSPLIT):
            pltpu.make_async_copy(
                rc_hbm.at[pl.ds(sp * RS, RS), pl.ds(i * CB, CB)],
                rc_buf.at[slot, pl.ds(sp * RS, RS), :],
                sems.at[slot, sp]).wait()
        s_ref[...] = _score(rc_buf[slot], ruleta_ref, sqrow_ref, w1g_ref,
                            bb2_ref, w2t_ref, b2_ref)

    @pl.when(i == NMAIN)
    def _():
        s_ref[0:1, 0:CTAIL] = _score(rc_tail_ref[...], ruleta_ref, sqrow_ref,
                                     w1g_ref, bb2_ref, w2t_ref, b2_ref)

    a = csa_ref[...]
    sidx_ref[...] = jnp.where(a != csb_ref[...], a, SENT)


def _dense_stage(rule_count, cs, rule_emb, rel_emb, ln_g, ln_b, W1, b1, W2,
                 b2):
    cs_next = jnp.concatenate([cs[1:], jnp.full((1,), -1, jnp.int32)])

    ruleta = jnp.zeros((24, R), jnp.float32)
    ruleta = ruleta.at[0:H].set(rule_emb.T)
    ruleta = ruleta.at[H].set(1.0)
    ruleta = ruleta.at[H + 1].set(jnp.mean(rule_emb.T, axis=0))
    sqrow = jnp.zeros((8, H), jnp.float32).at[0].set(1.0 / H)
    w1g = (W1[:H, :] * ln_g[:, None]).T
    bb2 = (W1[:H, :].T @ ln_b + W1[H:, :].T @ rel_emb + b1)
    bb2 = bb2.reshape(128, 1)
    w2t = jnp.zeros((8, 128), jnp.float32).at[0].set(W2[:, 0])

    rc_tail = rule_count[:, NMAIN * CB:]

    grid = (NMAIN + 1,)
    dense = pl.pallas_call(
        _dense_body,
        grid=grid,
        in_specs=[
            pl.BlockSpec(memory_space=pl.ANY),
            pl.BlockSpec((R, CTAIL), lambda i: (0, 0)),
            pl.BlockSpec((1, CB), lambda i: (0, i)),
            pl.BlockSpec((1, CB), lambda i: (0, i)),
            pl.BlockSpec((24, R), lambda i: (0, 0)),
            pl.BlockSpec((8, H), lambda i: (0, 0)),
            pl.BlockSpec((128, H), lambda i: (0, 0)),
            pl.BlockSpec((128, 1), lambda i: (0, 0)),
            pl.BlockSpec((8, 128), lambda i: (0, 0)),
            pl.BlockSpec((1, 1), lambda i: (0, 0)),
        ],
        out_specs=[
            pl.BlockSpec((1, CB), lambda i: (0, i)),
            pl.BlockSpec((1, CB), lambda i: (0, i)),
        ],
        out_shape=[
            jax.ShapeDtypeStruct((1, C), jnp.float32),
            jax.ShapeDtypeStruct((1, C), jnp.int32),
        ],
        scratch_shapes=[
            pltpu.VMEM((DEPTH, R, CB), jnp.float32),
            pltpu.SemaphoreType.DMA((DEPTH, NSPLIT)),
        ],
    )
    s2, sidx2 = dense(
        rule_count,
        rc_tail,
        cs.reshape(1, C),
        cs_next.reshape(1, C),
        ruleta,
        sqrow,
        w1g,
        bb2,
        w2t,
        b2.reshape(1, 1),
    )
    return s2, sidx2


def _lookup(starts_v, k):
    res = jnp.int32(0)
    io = lax.iota(jnp.int32, LANES)
    for v in range(NSTARTS // LANES):
        vec = starts_v[pl.ds(v * LANES, LANES)]
        sel = jnp.where(io == (k - v * LANES), vec, 0)
        res = res + jnp.sum(sel)
    return res


def _scatter_body(sidx_hbm, sval_hbm, bias_hbm, starts_hbm, out_hbm,
                  rowbuf, idx_v, val_v, starts_v):
    w = lax.axis_index("s") * NC + lax.axis_index("c")
    pltpu.sync_copy(starts_hbm, starts_v)
    for rr in range(2):
        r = w * 2 + rr
        row_base = r * E
        lo = _lookup(starts_v, r)
        hi = _lookup(starts_v, r + 1)
        pltpu.sync_copy(bias_hbm, rowbuf)
        lo8 = (lo // 8) * 8
        nch = (hi - lo8 + (CHUNK - 1)) // CHUNK

        def chunk_body(i, _, row_base=row_base, lo8=lo8):
            off = pl.multiple_of(lo8 + i * CHUNK, 8)
            pltpu.sync_copy(sidx_hbm.at[pl.ds(off, CHUNK)], idx_v)
            pltpu.sync_copy(sval_hbm.at[pl.ds(off, CHUNK)], val_v)

            def vec_body(j, __, row_base=row_base):
                idx = idx_v[pl.ds(j * LANES, LANES)]
                vals = val_v[pl.ds(j * LANES, LANES)]
                m = (idx >= row_base) & (idx < row_base + E)
                local = jnp.where(m, idx - row_base, 0)
                plsc.addupdate_scatter(rowbuf, [local], vals, mask=m)
                return 0

            lax.fori_loop(0, CHUNK // LANES, vec_body, 0)
            return 0

        lax.fori_loop(0, nch, chunk_body, 0)
        pltpu.sync_copy(rowbuf,
                        out_hbm.at[pl.ds(pl.multiple_of(row_base, 8), E)])


def kernel(rule_count, candidate_set, rule_emb, rel_emb, ln_g, ln_b,
           W1, b1, W2, b2, bias):
    cs = candidate_set
    s2, sidx2 = _dense_stage(rule_count, cs, rule_emb, rel_emb, ln_g, ln_b,
                             W1, b1, W2, b2)

    sidx_pad = jnp.concatenate(
        [sidx2.reshape(C), jnp.full((LPAD - C,), SENT, jnp.int32)])
    sval_pad = jnp.concatenate(
        [s2.reshape(C), jnp.zeros((LPAD - C,), jnp.float32)])

    bounds = jnp.arange(B + 1, dtype=jnp.int32) * E
    starts = jnp.searchsorted(cs, bounds).astype(jnp.int32)
    starts = jnp.concatenate(
        [starts, jnp.full((NSTARTS - (B + 1),), C, jnp.int32)])

    mesh = plsc.VectorSubcoreMesh(core_axis_name="c", subcore_axis_name="s")
    scatter = pl.kernel(
        _scatter_body,
        out_type=jax.ShapeDtypeStruct((B * E,), jnp.float32),
        mesh=mesh,
        scratch_types=[
            pltpu.VMEM((E,), jnp.float32),
            pltpu.VMEM((CHUNK,), jnp.int32),
            pltpu.VMEM((CHUNK,), jnp.float32),
            pltpu.VMEM((NSTARTS,), jnp.int32),
        ],
        compiler_params=pltpu.CompilerParams(needs_layout_passes=False),
    )
    flat = scatter(sidx_pad, sval_pad, bias, starts)

    score = flat.reshape(B, E)
    mask = jnp.ones((B, E), dtype=bool)
    return (score, mask)


# speedup vs baseline: 1.3084x; 1.3084x over previous
import jax
import jax.numpy as jnp
from jax import lax
from jax.experimental import pallas as pl
from jax.experimental.pallas import tpu as pltpu
from jax.experimental.pallas import tpu_sc as plsc

B = 64
E = 100000
R = 32
H = 16
C = 500000
EPS = 1e-6

CB = 16384
NMAIN = C // CB
CTAIL = C - NMAIN * CB
NBLK = NMAIN + 1
DEPTH = 4
NSPLIT = 4
RS = R // NSPLIT
CQ = CB // 8

SENT = B * E
CHUNK = 2048
LPAD = C + CHUNK + 16
NC = 2
NS = 16
LANES = 16
NSTARTS = 80


def _score(rc, ruleta_ref, sqrow_ref, w1g_ref, bb2_ref, w2t_ref, b2_ref):
    msgx = jnp.dot(ruleta_ref[...], rc,
                   preferred_element_type=jnp.float32)
    msg = msgx[0:H]
    d = msgx[H:H + 1] + EPS
    mu = msgx[H + 1:H + 2]
    xc = msg - mu
    var = jnp.dot(sqrow_ref[...], xc * xc,
                  preferred_element_type=jnp.float32)[0:1]
    inv = lax.rsqrt(var + 1e-5 * d * d)
    xhat = xc * inv
    z = jnp.dot(w1g_ref[...], xhat,
                preferred_element_type=jnp.float32)
    hdn = jnp.maximum(z + bb2_ref[...], 0.0)
    s = jnp.dot(w2t_ref[...], hdn,
                preferred_element_type=jnp.float32)[0:1]
    return s + b2_ref[...]


def _dense_body(rc_hbm, rc_tail_ref, ruleta_ref, sqrow_ref, w1g_ref, bb2_ref,
                w2t_ref, b2_ref, s_ref, rc_buf, sems):
    i = pl.program_id(0)

    def issue(slot, blk):
        for sp in range(NSPLIT):
            pltpu.make_async_copy(
                rc_hbm.at[pl.ds(sp * RS, RS), pl.ds(blk * CB, CB)],
                rc_buf.at[slot, pl.ds(sp * RS, RS), :],
                sems.at[slot, sp]).start()

    def wait(slot, blk):
        for sp in range(NSPLIT):
            pltpu.make_async_copy(
                rc_hbm.at[pl.ds(sp * RS, RS), pl.ds(blk * CB, CB)],
                rc_buf.at[slot, pl.ds(sp * RS, RS), :],
                sems.at[slot, sp]).wait()

    @pl.when(i == 0)
    def _():
        for b in range(DEPTH - 1):
            issue(b, b)

    nxt = i + DEPTH - 1

    @pl.when(nxt < NMAIN)
    def _():
        issue(lax.rem(nxt, DEPTH), nxt)

    @pl.when(i < NMAIN)
    def _():
        slot = lax.rem(i, DEPTH)
        wait(slot, i)
        s = _score(rc_buf[slot], ruleta_ref, sqrow_ref, w1g_ref, bb2_ref,
                   w2t_ref, b2_ref)
        s_ref[...] = jnp.reshape(s, (8, CQ))

    @pl.when(i == NMAIN)
    def _():
        st = _score(rc_tail_ref[...], ruleta_ref, sqrow_ref, w1g_ref,
                    bb2_ref, w2t_ref, b2_ref)
        sfull = jnp.concatenate(
            [st, jnp.zeros((1, CB - CTAIL), jnp.float32)], axis=1)
        s_ref[...] = jnp.reshape(sfull, (8, CQ))


def _dense_stage(rule_count, rule_emb, rel_emb, ln_g, ln_b, W1, b1, W2, b2):
    ruleta = jnp.zeros((24, R), jnp.float32)
    ruleta = ruleta.at[0:H].set(rule_emb.T)
    ruleta = ruleta.at[H].set(1.0)
    ruleta = ruleta.at[H + 1].set(jnp.mean(rule_emb.T, axis=0))
    sqrow = jnp.zeros((8, H), jnp.float32).at[0].set(1.0 / H)
    w1g = (W1[:H, :] * ln_g[:, None]).T
    bb2 = (W1[:H, :].T @ ln_b + W1[H:, :].T @ rel_emb + b1).reshape(128, 1)
    w2t = jnp.zeros((8, 128), jnp.float32).at[0].set(W2[:, 0])

    rc_tail = rule_count[:, NMAIN * CB:]

    s8 = pl.pallas_call(
        _dense_body,
        grid=(NBLK,),
        in_specs=[
            pl.BlockSpec(memory_space=pl.ANY),
            pl.BlockSpec((R, CTAIL), lambda i: (0, 0)),
            pl.BlockSpec((24, R), lambda i: (0, 0)),
            pl.BlockSpec((8, H), lambda i: (0, 0)),
            pl.BlockSpec((128, H), lambda i: (0, 0)),
            pl.BlockSpec((128, 1), lambda i: (0, 0)),
            pl.BlockSpec((8, 128), lambda i: (0, 0)),
            pl.BlockSpec((1, 1), lambda i: (0, 0)),
        ],
        out_specs=pl.BlockSpec((8, CQ), lambda i: (0, i)),
        out_shape=jax.ShapeDtypeStruct((8, CQ * NBLK), jnp.float32),
        scratch_shapes=[
            pltpu.VMEM((DEPTH, R, CB), jnp.float32),
            pltpu.SemaphoreType.DMA((DEPTH, NSPLIT)),
        ],
    )(rule_count, rc_tail, ruleta, sqrow, w1g, bb2, w2t, b2.reshape(1, 1))

    s_lin = s8.reshape(8, NBLK, CQ).transpose(1, 0, 2).reshape(NBLK * CB)
    return s_lin[:C]


def _lookup(starts_v, k):
    res = jnp.int32(0)
    io = lax.iota(jnp.int32, LANES)
    for v in range(NSTARTS // LANES):
        vec = starts_v[pl.ds(v * LANES, LANES)]
        sel = jnp.where(io == (k - v * LANES), vec, 0)
        res = res + jnp.sum(sel)
    return res


def _scatter_body(cs_hbm, sval_hbm, bias_hbm, starts_hbm, out_hbm,
                  rowbuf, idx_v, val_v, starts_v):
    w = lax.axis_index("s") * NC + lax.axis_index("c")
    pltpu.sync_copy(starts_hbm, starts_v)
    for rr in range(2):
        r = w * 2 + rr
        row_base = r * E
        lo = _lookup(starts_v, r)
        hi = _lookup(starts_v, r + 1)
        pltpu.sync_copy(bias_hbm, rowbuf)
        lo8 = (lo // 8) * 8
        nch = (hi - lo8 + (CHUNK - 1)) // CHUNK

        def chunk_body(i, _, row_base=row_base, lo8=lo8):
            off = pl.multiple_of(lo8 + i * CHUNK, 8)
            pltpu.sync_copy(cs_hbm.at[pl.ds(off, CHUNK + LANES)], idx_v)
            pltpu.sync_copy(sval_hbm.at[pl.ds(off, CHUNK)], val_v)

            def vec_body(j, __, row_base=row_base):
                idx = idx_v[pl.ds(j * LANES, LANES)]
                nxt = idx_v[pl.ds(j * LANES + 1, LANES)]
                vals = val_v[pl.ds(j * LANES, LANES)]
                m = (idx >= row_base) & (idx < row_base + E) & (idx != nxt)
                local = jnp.where(m, idx - row_base, 0)
                plsc.addupdate_scatter(rowbuf, [local], vals, mask=m)
                return 0

            lax.fori_loop(0, CHUNK // LANES, vec_body, 0)
            return 0

        lax.fori_loop(0, nch, chunk_body, 0)
        pltpu.sync_copy(rowbuf,
                        out_hbm.at[pl.ds(pl.multiple_of(row_base, 8), E)])


def kernel(rule_count, candidate_set, rule_emb, rel_emb, ln_g, ln_b,
           W1, b1, W2, b2, bias):
    cs = candidate_set
    s_lin = _dense_stage(rule_count, rule_emb, rel_emb, ln_g, ln_b, W1, b1,
                         W2, b2)

    cs_pad = jnp.concatenate([cs, jnp.full((LPAD - C,), SENT, jnp.int32)])
    sval_pad = jnp.concatenate([s_lin, jnp.zeros((LPAD - C,), jnp.float32)])

    bounds = jnp.arange(B + 1, dtype=jnp.int32) * E
    starts = jnp.searchsorted(cs, bounds).astype(jnp.int32)
    starts = jnp.concatenate(
        [starts, jnp.full((NSTARTS - (B + 1),), C, jnp.int32)])

    mesh = plsc.VectorSubcoreMesh(core_axis_name="c", subcore_axis_name="s")
    scatter = pl.kernel(
        _scatter_body,
        out_type=jax.ShapeDtypeStruct((B * E,), jnp.float32),
        mesh=mesh,
        scratch_types=[
            pltpu.VMEM((E,), jnp.float32),
            pltpu.VMEM((CHUNK + LANES,), jnp.int32),
            pltpu.VMEM((CHUNK,), jnp.float32),
            pltpu.VMEM((NSTARTS,), jnp.int32),
        ],
        compiler_params=pltpu.CompilerParams(needs_layout_passes=False),
    )
    flat = scatter(cs_pad, sval_pad, bias, starts)

    score = flat.reshape(B, E)
    mask = jnp.ones((B, E), dtype=bool)
    return (score, mask)


# speedup vs baseline: 1.3376x; 1.0223x over previous
import jax
import jax.numpy as jnp
from jax import lax
from jax.experimental import pallas as pl
from jax.experimental.pallas import tpu as pltpu
from jax.experimental.pallas import tpu_sc as plsc

B = 64
E = 100000
R = 32
H = 16
C = 500000
EPS = 1e-6

CB = 16384
NMAIN = C // CB
CTAIL = C - NMAIN * CB
NBLK = NMAIN + 1
DEPTH = 4
NSPLIT = 4
RS = R // NSPLIT
CQ = CB // 8

SENT = B * E
CHUNK = 4096
LPAD = C + CHUNK + 16
NC = 2
NS = 16
LANES = 16
NSTARTS = 80


def _score(rc, ruleta_ref, sqrow_ref, w1g_ref, bb2_ref, w2t_ref, b2_ref):
    msgx = jnp.dot(ruleta_ref[...], rc,
                   preferred_element_type=jnp.float32)
    msg = msgx[0:H]
    d = msgx[H:H + 1] + EPS
    mu = msgx[H + 1:H + 2]
    xc = msg - mu
    var = jnp.dot(sqrow_ref[...], xc * xc,
                  preferred_element_type=jnp.float32)[0:1]
    inv = lax.rsqrt(var + 1e-5 * d * d)
    xhat = xc * inv
    z = jnp.dot(w1g_ref[...], xhat,
                preferred_element_type=jnp.float32)
    hdn = jnp.maximum(z + bb2_ref[...], 0.0)
    s = jnp.dot(w2t_ref[...], hdn,
                preferred_element_type=jnp.float32)[0:1]
    return s + b2_ref[...]


def _dense_body(rc_hbm, rc_tail_ref, ruleta_ref, sqrow_ref, w1g_ref, bb2_ref,
                w2t_ref, b2_ref, s_ref, rc_buf, sems):
    i = pl.program_id(0)

    def issue(slot, blk):
        for sp in range(NSPLIT):
            pltpu.make_async_copy(
                rc_hbm.at[pl.ds(sp * RS, RS), pl.ds(blk * CB, CB)],
                rc_buf.at[slot, pl.ds(sp * RS, RS), :],
                sems.at[slot, sp]).start()

    def wait(slot, blk):
        for sp in range(NSPLIT):
            pltpu.make_async_copy(
                rc_hbm.at[pl.ds(sp * RS, RS), pl.ds(blk * CB, CB)],
                rc_buf.at[slot, pl.ds(sp * RS, RS), :],
                sems.at[slot, sp]).wait()

    @pl.when(i == 0)
    def _():
        for b in range(DEPTH - 1):
            issue(b, b)

    nxt = i + DEPTH - 1

    @pl.when(nxt < NMAIN)
    def _():
        issue(lax.rem(nxt, DEPTH), nxt)

    @pl.when(i < NMAIN)
    def _():
        slot = lax.rem(i, DEPTH)
        wait(slot, i)
        s = _score(rc_buf[slot], ruleta_ref, sqrow_ref, w1g_ref, bb2_ref,
                   w2t_ref, b2_ref)
        s_ref[...] = jnp.reshape(s, (8, CQ))

    @pl.when(i == NMAIN)
    def _():
        st = _score(rc_tail_ref[...], ruleta_ref, sqrow_ref, w1g_ref,
                    bb2_ref, w2t_ref, b2_ref)
        sfull = jnp.concatenate(
            [st, jnp.zeros((1, CB - CTAIL), jnp.float32)], axis=1)
        s_ref[...] = jnp.reshape(sfull, (8, CQ))


def _dense_stage(rule_count, rule_emb, rel_emb, ln_g, ln_b, W1, b1, W2, b2):
    ruleta = jnp.zeros((24, R), jnp.float32)
    ruleta = ruleta.at[0:H].set(rule_emb.T)
    ruleta = ruleta.at[H].set(1.0)
    ruleta = ruleta.at[H + 1].set(jnp.mean(rule_emb.T, axis=0))
    sqrow = jnp.zeros((8, H), jnp.float32).at[0].set(1.0 / H)
    w1g = (W1[:H, :] * ln_g[:, None]).T
    bb2 = (W1[:H, :].T @ ln_b + W1[H:, :].T @ rel_emb + b1).reshape(128, 1)
    w2t = jnp.zeros((8, 128), jnp.float32).at[0].set(W2[:, 0])

    rc_tail = rule_count[:, NMAIN * CB:]

    s8 = pl.pallas_call(
        _dense_body,
        grid=(NBLK,),
        in_specs=[
            pl.BlockSpec(memory_space=pl.ANY),
            pl.BlockSpec((R, CTAIL), lambda i: (0, 0)),
            pl.BlockSpec((24, R), lambda i: (0, 0)),
            pl.BlockSpec((8, H), lambda i: (0, 0)),
            pl.BlockSpec((128, H), lambda i: (0, 0)),
            pl.BlockSpec((128, 1), lambda i: (0, 0)),
            pl.BlockSpec((8, 128), lambda i: (0, 0)),
            pl.BlockSpec((1, 1), lambda i: (0, 0)),
        ],
        out_specs=pl.BlockSpec((8, CQ), lambda i: (0, i)),
        out_shape=jax.ShapeDtypeStruct((8, CQ * NBLK), jnp.float32),
        scratch_shapes=[
            pltpu.VMEM((DEPTH, R, CB), jnp.float32),
            pltpu.SemaphoreType.DMA((DEPTH, NSPLIT)),
        ],
    )(rule_count, rc_tail, ruleta, sqrow, w1g, bb2, w2t, b2.reshape(1, 1))

    s_lin = s8.reshape(8, NBLK, CQ).transpose(1, 0, 2).reshape(NBLK * CB)
    return s_lin[:C]


def _lookup(starts_v, k):
    res = jnp.int32(0)
    io = lax.iota(jnp.int32, LANES)
    for v in range(NSTARTS // LANES):
        vec = starts_v[pl.ds(v * LANES, LANES)]
        sel = jnp.where(io == (k - v * LANES), vec, 0)
        res = res + jnp.sum(sel)
    return res


def _scatter_body(cs_hbm, sval_hbm, bias_hbm, starts_hbm, out_hbm,
                  rowbuf, idx_v, val_v, starts_v):
    w = lax.axis_index("s") * NC + lax.axis_index("c")
    pltpu.sync_copy(starts_hbm, starts_v)
    for rr in range(2):
        r = w * 2 + rr
        row_base = r * E
        lo = _lookup(starts_v, r)
        hi = _lookup(starts_v, r + 1)
        pltpu.sync_copy(bias_hbm, rowbuf)
        lo8 = (lo // 8) * 8
        nch = (hi - lo8 + (CHUNK - 1)) // CHUNK

        def chunk_body(i, _, row_base=row_base, lo8=lo8):
            off = pl.multiple_of(lo8 + i * CHUNK, 8)
            pltpu.sync_copy(cs_hbm.at[pl.ds(off, CHUNK + LANES)], idx_v)
            pltpu.sync_copy(sval_hbm.at[pl.ds(off, CHUNK)], val_v)

            def vec_body(j, __, row_base=row_base):
                idx = idx_v[pl.ds(j * LANES, LANES)]
                nxt = idx_v[pl.ds(j * LANES + 1, LANES)]
                vals = val_v[pl.ds(j * LANES, LANES)]
                m = (idx >= row_base) & (idx < row_base + E) & (idx != nxt)
                local = jnp.where(m, idx - row_base, 0)
                plsc.addupdate_scatter(rowbuf, [local], vals, mask=m)
                return 0

            lax.fori_loop(0, CHUNK // LANES, vec_body, 0, unroll=8)
            return 0

        lax.fori_loop(0, nch, chunk_body, 0)
        pltpu.sync_copy(rowbuf,
                        out_hbm.at[pl.ds(pl.multiple_of(row_base, 8), E)])


def kernel(rule_count, candidate_set, rule_emb, rel_emb, ln_g, ln_b,
           W1, b1, W2, b2, bias):
    cs = candidate_set
    s_lin = _dense_stage(rule_count, rule_emb, rel_emb, ln_g, ln_b, W1, b1,
                         W2, b2)

    cs_pad = jnp.concatenate([cs, jnp.full((LPAD - C,), SENT, jnp.int32)])
    sval_pad = jnp.concatenate([s_lin, jnp.zeros((LPAD - C,), jnp.float32)])

    bounds = jnp.arange(B + 1, dtype=jnp.int32) * E
    starts = jnp.searchsorted(cs, bounds).astype(jnp.int32)
    starts = jnp.concatenate(
        [starts, jnp.full((NSTARTS - (B + 1),), C, jnp.int32)])

    mesh = plsc.VectorSubcoreMesh(core_axis_name="c", subcore_axis_name="s")
    scatter = pl.kernel(
        _scatter_body,
        out_type=jax.ShapeDtypeStruct((B * E,), jnp.float32),
        mesh=mesh,
        scratch_types=[
            pltpu.VMEM((E,), jnp.float32),
            pltpu.VMEM((CHUNK + LANES,), jnp.int32),
            pltpu.VMEM((CHUNK,), jnp.float32),
            pltpu.VMEM((NSTARTS,), jnp.int32),
        ],
        compiler_params=pltpu.CompilerParams(needs_layout_passes=False),
    )
    flat = scatter(cs_pad, sval_pad, bias, starts)

    score = flat.reshape(B, E)
    mask = jnp.ones((B, E), dtype=bool)
    return (score, mask)
